# Initial kernel scaffold; baseline (speedup 1.0000x reference)
import jax
import jax.numpy as jnp
from jax.experimental import pallas as pl


def kernel(idx, tok_emb, pos_emb, Wk, Wq, Wv, Wproj, bproj, ln1_g, ln1_b, ln2_g, ln2_b, Wroute, broute, Wnoise, bnoise, We1, be1, We2, be2, eln_g, eln_b, lnf_g, lnf_b, Wlm, blm):
    raise NotImplementedError("write your pallas kernel here")


# speedup vs baseline: 1.3705x; 1.3705x over previous
import jax
import jax.numpy as jnp
from jax.experimental import pallas as pl
from jax.experimental.pallas import tpu as pltpu
from jax.experimental.pallas import tpu_sc as plsc

B = 32
T = 48
V = 50257
D = 144
L = 6
H = 6
HS = 24
E = 6
FF = 576
NT = B * T


def _ln(x, g, b):
    m = jnp.mean(x, axis=-1, keepdims=True)
    v = jnp.mean((x - m) ** 2, axis=-1, keepdims=True)
    return (x - m) / jnp.sqrt(v + 1e-5) * g + b


def _sc_gather(tok_emb, idx_flat):
    mesh = plsc.VectorSubcoreMesh(core_axis_name="core", subcore_axis_name="subcore")
    W = 48

    @pl.kernel(out_type=jax.ShapeDtypeStruct((NT, 256), jnp.float32), mesh=mesh)
    def gk(emb_hbm, i_hbm, o_hbm):
        def body(i_vmem, o_vmem):
            pltpu.sync_copy(emb_hbm.at[i_vmem.at[0]], o_vmem)

        pltpu.emit_pipeline(
            body,
            grid=(NT // W,),
            in_specs=[pl.BlockSpec((1, W), index_map=lambda i: (i, 0))],
            out_specs=[pl.BlockSpec((W, 256), index_map=lambda i: (i, 0))],
            core_axis_name=("core", "subcore"),
            dimension_semantics=(pltpu.PARALLEL,),
        )(i_hbm, o_hbm)

    return gk(tok_emb, idx_flat)


def _layers_body(xg_ref, pos_ref, nrm_ref,
                 Wqt_ref, Wkt_ref, Wvt_ref, Wproj_ref, bproj_ref,
                 ln1g_ref, ln1b_ref, ln2g_ref, ln2b_ref,
                 Wroute_ref, broute_ref, Wnoise_ref, bnoise_ref,
                 We1_ref, be1_ref, We2_ref, be2_ref, elng_ref, elnb_ref,
                 lnfg_ref, lnfb_ref,
                 xn_ref, aux_ref, x_s, att_s, q_s, k_s, v_s):
    l = pl.program_id(0)

    @pl.when(l == 0)
    def _():
        pos = pos_ref[...]
        x0 = xg_ref[...][:, :D].reshape(B, T, D) + pos[None, :, :]
        x_s[...] = x0.reshape(NT, D)
        aux_ref[0, 0] = 0.0


    x = x_s[...]

    h = _ln(x, ln1g_ref[0, 0, :], ln1b_ref[0, 0, :])
    q_s[...] = jnp.dot(h, Wqt_ref[0])
    k_s[...] = jnp.dot(h, Wkt_ref[0])
    v_s[...] = jnp.dot(h, Wvt_ref[0])

    d_i = jax.lax.broadcasted_iota(jnp.int32, (D, H * T), 0)
    c_i = jax.lax.broadcasted_iota(jnp.int32, (D, H * T), 1)
    mask_k = (d_i // HS) == (c_i // T)
    s_i = jax.lax.broadcasted_iota(jnp.int32, (H * T, D), 0)
    e_i = jax.lax.broadcasted_iota(jnp.int32, (H * T, D), 1)
    mask_v = (s_i // T) == (e_i // HS)
    t_i = jax.lax.broadcasted_iota(jnp.int32, (T, H * T), 0)
    u_i = jax.lax.broadcasted_iota(jnp.int32, (T, H * T), 1)
    causal = (u_i % T) <= t_i

    def att_batch(b, carry):
        qb = q_s[pl.ds(b * T, T), :]
        kb = k_s[pl.ds(b * T, T), :]
        vb = v_s[pl.ds(b * T, T), :]
        kt = jnp.transpose(kb)
        ktil = jnp.where(mask_k, jnp.concatenate([kt] * H, axis=1), 0.0)
        wei = jnp.dot(qb, ktil) * (D ** -0.5)
        wei = jnp.where(causal, wei, -jnp.inf)
        w3 = wei.reshape(T, H, T)
        w3 = w3 - jnp.max(w3, axis=-1, keepdims=True)
        e3 = jnp.exp(w3)
        p3 = e3 / jnp.sum(e3, axis=-1, keepdims=True)
        p = p3.reshape(T, H * T)
        vtil = jnp.where(mask_v, jnp.concatenate([vb] * H, axis=0), 0.0)
        att_s[pl.ds(b * T, T), :] = jnp.dot(p, vtil)
        return carry

    jax.lax.fori_loop(0, B, att_batch, 0)
    x = x + jnp.dot(att_s[...], Wproj_ref[0]) + bproj_ref[0, 0, :]

    flat = _ln(x, ln2g_ref[0, 0, :], ln2b_ref[0, 0, :])
    logits = jnp.dot(flat, Wroute_ref[0]) + broute_ref[0, 0, :]
    nlog = jnp.dot(flat, Wnoise_ref[0]) + bnoise_ref[0, 0, :]
    noisy = logits + nrm_ref[0] * jax.nn.softplus(nlog)

    eidx = jax.lax.broadcasted_iota(jnp.int32, (NT, E), 1)
    m1 = jnp.max(noisy, axis=-1, keepdims=True)
    i1 = jnp.argmax(noisy, axis=-1)[:, None]
    noisy2 = jnp.where(eidx == i1, -jnp.inf, noisy)
    i2 = jnp.argmax(noisy2, axis=-1)[:, None]
    sel = (eidx == i1) | (eidx == i2)
    ex = jnp.where(sel, jnp.exp(noisy - m1), 0.0)
    gating = ex / jnp.sum(ex, axis=-1, keepdims=True)

    counts = jnp.sum(sel.astype(jnp.float32), axis=0)
    probs = (counts + 0.001) / (jnp.float32(2 * NT) + 0.001 * E)
    u = jnp.float32(1.0 / E)
    aux = jnp.sum(u * (jnp.log(u) - jnp.log(probs + 1e-10))) / E
    aux_ref[0, 0] = aux_ref[0, 0] + 0.02 * aux

    moe = jnp.zeros((NT, D), jnp.float32)
    for e in range(E):
        h1 = jnp.dot(flat, We1_ref[0, e]) + be1_ref[0, e, :]
        h1 = 0.5 * h1 * (1.0 + jax.lax.erf(h1 * jnp.float32(0.7071067811865476)))
        eo = jnp.dot(h1, We2_ref[0, e]) + be2_ref[0, e, :]
        eo = _ln(flat + eo, elng_ref[0, e, :], elnb_ref[0, e, :])
        moe = moe + eo * gating[:, e][:, None]
    x = x + moe
    x_s[...] = x

    @pl.when(l == L - 1)
    def _():
        xn_ref[...] = _ln(x, lnfg_ref[...], lnfb_ref[...])


def _layers_call(xg, pos_emb, normals, Wqt, Wkt, Wvt, Wproj, bproj,
                 ln1_g, ln1_b, ln2_g, ln2_b, Wroute, broute, Wnoise, bnoise,
                 We1, be1, We2, be2, eln_g, eln_b, lnf_g, lnf_b):
    per_l = lambda blk: pl.BlockSpec(blk, lambda l: (l,) + (0,) * (len(blk) - 1))
    full = lambda shp: pl.BlockSpec(shp, lambda l: (0,) * len(shp))
    return pl.pallas_call(
        _layers_body,
        grid=(L,),
        in_specs=[
            full((NT, 256)), full((T, D)), per_l((1, NT, E)),
            per_l((1, D, D)), per_l((1, D, D)), per_l((1, D, D)),
            per_l((1, D, D)), per_l((1, 1, D)),
            per_l((1, 1, D)), per_l((1, 1, D)), per_l((1, 1, D)), per_l((1, 1, D)),
            per_l((1, D, E)), per_l((1, 1, E)), per_l((1, D, E)), per_l((1, 1, E)),
            per_l((1, E, D, FF)), per_l((1, E, FF)),
            per_l((1, E, FF, D)), per_l((1, E, D)),
            per_l((1, E, D)), per_l((1, E, D)),
            full((D,)), full((D,)),
        ],
        out_specs=[full((NT, D)),
                   pl.BlockSpec((1, 1), lambda l: (0, 0),
                                memory_space=pltpu.SMEM)],
        out_shape=[jax.ShapeDtypeStruct((NT, D), jnp.float32),
                   jax.ShapeDtypeStruct((1, 1), jnp.float32)],
        scratch_shapes=[pltpu.VMEM((NT, D), jnp.float32)] * 5,
    )(xg, pos_emb, normals, Wqt, Wkt, Wvt, Wproj, bproj,
      ln1_g, ln1_b, ln2_g, ln2_b, Wroute, broute, Wnoise, bnoise,
      We1, be1, We2, be2, eln_g, eln_b, lnf_g, lnf_b)


VB = 2048


def _lm_body(xn_ref, wlm_ref, blm_ref, out_ref):
    out_ref[...] = jnp.dot(xn_ref[...], wlm_ref[...]) + blm_ref[0, :]


def _lm_call(xn, Wlm, blm):
    nvb = pl.cdiv(V, VB)
    return pl.pallas_call(
        _lm_body,
        grid=(nvb,),
        in_specs=[
            pl.BlockSpec((NT, D), lambda j: (0, 0)),
            pl.BlockSpec((D, VB), lambda j: (0, j)),
            pl.BlockSpec((1, VB), lambda j: (0, j)),
        ],
        out_specs=pl.BlockSpec((NT, VB), lambda j: (0, j)),
        out_shape=jax.ShapeDtypeStruct((NT, V), jnp.float32),
    )(xn, Wlm, blm)


def kernel(idx, tok_emb, pos_emb, Wk, Wq, Wv, Wproj, bproj, ln1_g, ln1_b,
           ln2_g, ln2_b, Wroute, broute, Wnoise, bnoise, We1, be1, We2, be2,
           eln_g, eln_b, lnf_g, lnf_b, Wlm, blm):
    idx_flat = idx.reshape(NT // 48, 48).astype(jnp.int32)
    temb_p = jnp.pad(tok_emb, ((0, 0), (0, 256 - D)))
    xg = _sc_gather(temb_p, idx_flat)

    nkey = jax.random.key(42)
    normals = jnp.stack([
        jax.random.normal(jax.random.fold_in(nkey, l), (NT, E), dtype=jnp.float32)
        for l in range(L)
    ])

    Wqt = Wq.transpose(0, 2, 1, 3).reshape(L, D, H * HS)
    Wkt = Wk.transpose(0, 2, 1, 3).reshape(L, D, H * HS)
    Wvt = Wv.transpose(0, 2, 1, 3).reshape(L, D, H * HS)

    r3 = lambda a: a.reshape(L, 1, -1)
    xn, aux = _layers_call(xg, pos_emb, normals, Wqt, Wkt, Wvt, Wproj, r3(bproj),
                           r3(ln1_g), r3(ln1_b), r3(ln2_g), r3(ln2_b), Wroute,
                           r3(broute), Wnoise, r3(bnoise), We1, be1, We2, be2,
                           eln_g, eln_b, lnf_g, lnf_b)
    out = _lm_call(xn, Wlm, blm.reshape(1, V))
    return out.reshape(B, T, V), aux[0, 0]


# speedup vs baseline: 1.8364x; 1.3399x over previous
import jax
import jax.numpy as jnp
from jax.experimental import pallas as pl
from jax.experimental.pallas import tpu as pltpu
from jax.experimental.pallas import tpu_sc as plsc

B = 32
T = 48
V = 50257
D = 144
L = 6
H = 6
HS = 24
E = 6
FF = 576
NT = B * T


def _ln(x, g, b):
    m = jnp.mean(x, axis=-1, keepdims=True)
    v = jnp.mean((x - m) ** 2, axis=-1, keepdims=True)
    return (x - m) / jnp.sqrt(v + 1e-5) * g + b


def _sc_gather(tok_emb, idx_flat):
    mesh = plsc.VectorSubcoreMesh(core_axis_name="core", subcore_axis_name="subcore")
    W = 48

    @pl.kernel(out_type=jax.ShapeDtypeStruct((NT, 256), jnp.float32), mesh=mesh)
    def gk(emb_hbm, i_hbm, o_hbm):
        def body(i_vmem, o_vmem):
            pltpu.sync_copy(emb_hbm.at[i_vmem.at[0]], o_vmem)

        pltpu.emit_pipeline(
            body,
            grid=(NT // W,),
            in_specs=[pl.BlockSpec((1, W), index_map=lambda i: (i, 0))],
            out_specs=[pl.BlockSpec((W, 256), index_map=lambda i: (i, 0))],
            core_axis_name=("core", "subcore"),
            dimension_semantics=(pltpu.PARALLEL,),
        )(i_hbm, o_hbm)

    return gk(tok_emb, idx_flat)


def _layers_body(xg_ref, pos_ref, nrm_ref,
                 Wqt_ref, Wkt_ref, Wvt_ref, Wproj_ref, bproj_ref,
                 ln1g_ref, ln1b_ref, ln2g_ref, ln2b_ref,
                 Wroute_ref, broute_ref, Wnoise_ref, bnoise_ref,
                 We1_ref, be1_ref, We2_ref, be2_ref, elng_ref, elnb_ref,
                 lnfg_ref, lnfb_ref,
                 xn_ref, aux_ref, x_s, att_s, q_s, k_s, v_s):
    l = pl.program_id(0)

    @pl.when(l == 0)
    def _():
        pos = pos_ref[...]
        x0 = xg_ref[...][:, :D].reshape(B, T, D) + pos[None, :, :]
        x_s[...] = x0.reshape(NT, D)
        aux_ref[0, 0] = 0.0


    x = x_s[...]

    h = _ln(x, ln1g_ref[0, 0, :], ln1b_ref[0, 0, :])
    q_s[...] = jnp.dot(h, Wqt_ref[0])
    k_s[...] = jnp.dot(h, Wkt_ref[0])
    v_s[...] = jnp.dot(h, Wvt_ref[0])

    d_i = jax.lax.broadcasted_iota(jnp.int32, (D, H * T), 0)
    c_i = jax.lax.broadcasted_iota(jnp.int32, (D, H * T), 1)
    mask_k = (d_i // HS) == (c_i // T)
    s_i = jax.lax.broadcasted_iota(jnp.int32, (H * T, D), 0)
    e_i = jax.lax.broadcasted_iota(jnp.int32, (H * T, D), 1)
    mask_v = (s_i // T) == (e_i // HS)
    t_i = jax.lax.broadcasted_iota(jnp.int32, (T, H * T), 0)
    u_i = jax.lax.broadcasted_iota(jnp.int32, (T, H * T), 1)
    causal = (u_i % T) <= t_i

    def att_batch(b, carry):
        qb = q_s[pl.ds(b * T, T), :]
        kb = k_s[pl.ds(b * T, T), :]
        vb = v_s[pl.ds(b * T, T), :]
        kt = jnp.transpose(kb)
        ktil = jnp.where(mask_k, jnp.concatenate([kt] * H, axis=1), 0.0)
        wei = jnp.dot(qb, ktil) * (D ** -0.5)
        wei = jnp.where(causal, wei, -jnp.inf)
        w3 = wei.reshape(T, H, T)
        w3 = w3 - jnp.max(w3, axis=-1, keepdims=True)
        e3 = jnp.exp(w3)
        p3 = e3 / jnp.sum(e3, axis=-1, keepdims=True)
        p = p3.reshape(T, H * T)
        vtil = jnp.where(mask_v, jnp.concatenate([vb] * H, axis=0), 0.0)
        att_s[pl.ds(b * T, T), :] = jnp.dot(p, vtil)
        return carry

    jax.lax.fori_loop(0, B, att_batch, 0)
    x = x + jnp.dot(att_s[...], Wproj_ref[0]) + bproj_ref[0, 0, :]

    flat = _ln(x, ln2g_ref[0, 0, :], ln2b_ref[0, 0, :])
    logits = jnp.dot(flat, Wroute_ref[0]) + broute_ref[0, 0, :]
    nlog = jnp.dot(flat, Wnoise_ref[0]) + bnoise_ref[0, 0, :]
    noisy = logits + nrm_ref[0] * jax.nn.softplus(nlog)

    eidx = jax.lax.broadcasted_iota(jnp.int32, (NT, E), 1)
    m1 = jnp.max(noisy, axis=-1, keepdims=True)
    i1 = jnp.argmax(noisy, axis=-1)[:, None]
    noisy2 = jnp.where(eidx == i1, -jnp.inf, noisy)
    i2 = jnp.argmax(noisy2, axis=-1)[:, None]
    sel = (eidx == i1) | (eidx == i2)
    ex = jnp.where(sel, jnp.exp(noisy - m1), 0.0)
    gating = ex / jnp.sum(ex, axis=-1, keepdims=True)

    counts = jnp.sum(sel.astype(jnp.float32), axis=0)
    probs = (counts + 0.001) / (jnp.float32(2 * NT) + 0.001 * E)
    u = jnp.float32(1.0 / E)
    aux = jnp.sum(u * (jnp.log(u) - jnp.log(probs + 1e-10))) / E
    aux_ref[0, 0] = aux_ref[0, 0] + 0.02 * aux

    moe = jnp.zeros((NT, D), jnp.float32)
    for e in range(E):
        h1 = jnp.dot(flat, We1_ref[0, e]) + be1_ref[0, e, :]
        h1 = 0.5 * h1 * (1.0 + jax.lax.erf(h1 * jnp.float32(0.7071067811865476)))
        eo = jnp.dot(h1, We2_ref[0, e]) + be2_ref[0, e, :]
        eo = _ln(flat + eo, elng_ref[0, e, :], elnb_ref[0, e, :])
        moe = moe + eo * gating[:, e][:, None]
    x = x + moe
    x_s[...] = x

    @pl.when(l == L - 1)
    def _():
        xn_ref[...] = _ln(x, lnfg_ref[...], lnfb_ref[...])


def _layers_call(xg, pos_emb, normals, Wqt, Wkt, Wvt, Wproj, bproj,
                 ln1_g, ln1_b, ln2_g, ln2_b, Wroute, broute, Wnoise, bnoise,
                 We1, be1, We2, be2, eln_g, eln_b, lnf_g, lnf_b):
    per_l = lambda blk: pl.BlockSpec(blk, lambda l: (l,) + (0,) * (len(blk) - 1))
    full = lambda shp: pl.BlockSpec(shp, lambda l: (0,) * len(shp))
    return pl.pallas_call(
        _layers_body,
        grid=(L,),
        in_specs=[
            full((NT, 256)), full((T, D)), per_l((1, NT, E)),
            per_l((1, D, D)), per_l((1, D, D)), per_l((1, D, D)),
            per_l((1, D, D)), per_l((1, 1, D)),
            per_l((1, 1, D)), per_l((1, 1, D)), per_l((1, 1, D)), per_l((1, 1, D)),
            per_l((1, D, E)), per_l((1, 1, E)), per_l((1, D, E)), per_l((1, 1, E)),
            per_l((1, E, D, FF)), per_l((1, E, FF)),
            per_l((1, E, FF, D)), per_l((1, E, D)),
            per_l((1, E, D)), per_l((1, E, D)),
            full((D,)), full((D,)),
        ],
        out_specs=[full((NT, D)),
                   pl.BlockSpec((1, 1), lambda l: (0, 0),
                                memory_space=pltpu.SMEM)],
        out_shape=[jax.ShapeDtypeStruct((NT, D), jnp.float32),
                   jax.ShapeDtypeStruct((1, 1), jnp.float32)],
        scratch_shapes=[pltpu.VMEM((NT, D), jnp.float32)] * 5,
    )(xg, pos_emb, normals, Wqt, Wkt, Wvt, Wproj, bproj,
      ln1_g, ln1_b, ln2_g, ln2_b, Wroute, broute, Wnoise, bnoise,
      We1, be1, We2, be2, eln_g, eln_b, lnf_g, lnf_b)


PR = 2048


def _pad_body(t_ref, o_ref):
    o_ref[:, :D] = t_ref[...]
    o_ref[:, D:] = jnp.zeros((PR, 256 - D), jnp.float32)


def _pad_call(tok_emb):
    return pl.pallas_call(
        _pad_body,
        grid=(pl.cdiv(V, PR),),
        in_specs=[pl.BlockSpec((PR, D), lambda i: (i, 0))],
        out_specs=pl.BlockSpec((PR, 256), lambda i: (i, 0)),
        out_shape=jax.ShapeDtypeStruct((V, 256), jnp.float32),
    )(tok_emb)


VB = 2048


def _lm_body(xn_ref, wlm_ref, blm_ref, out_ref):
    out_ref[...] = jnp.dot(xn_ref[...], wlm_ref[...]) + blm_ref[0, :]


def _lm_call(xn, Wlm, blm):
    nvb = pl.cdiv(V, VB)
    return pl.pallas_call(
        _lm_body,
        grid=(nvb,),
        in_specs=[
            pl.BlockSpec((NT, D), lambda j: (0, 0)),
            pl.BlockSpec((D, VB), lambda j: (0, j)),
            pl.BlockSpec((1, VB), lambda j: (0, j)),
        ],
        out_specs=pl.BlockSpec((NT, VB), lambda j: (0, j)),
        out_shape=jax.ShapeDtypeStruct((NT, V), jnp.float32),
    )(xn, Wlm, blm)


def kernel(idx, tok_emb, pos_emb, Wk, Wq, Wv, Wproj, bproj, ln1_g, ln1_b,
           ln2_g, ln2_b, Wroute, broute, Wnoise, bnoise, We1, be1, We2, be2,
           eln_g, eln_b, lnf_g, lnf_b, Wlm, blm):
    idx_flat = idx.reshape(NT // 48, 48).astype(jnp.int32)
    temb_p = _pad_call(tok_emb)
    xg = _sc_gather(temb_p, idx_flat)

    nkey = jax.random.key(42)
    normals = jnp.stack([
        jax.random.normal(jax.random.fold_in(nkey, l), (NT, E), dtype=jnp.float32)
        for l in range(L)
    ])

    Wqt = Wq.transpose(0, 2, 1, 3).reshape(L, D, H * HS)
    Wkt = Wk.transpose(0, 2, 1, 3).reshape(L, D, H * HS)
    Wvt = Wv.transpose(0, 2, 1, 3).reshape(L, D, H * HS)

    r3 = lambda a: a.reshape(L, 1, -1)
    xn, aux = _layers_call(xg, pos_emb, normals, Wqt, Wkt, Wvt, Wproj, r3(bproj),
                           r3(ln1_g), r3(ln1_b), r3(ln2_g), r3(ln2_b), Wroute,
                           r3(broute), Wnoise, r3(bnoise), We1, be1, We2, be2,
                           eln_g, eln_b, lnf_g, lnf_b)
    out = _lm_call(xn, Wlm, blm.reshape(1, V))
    return out.reshape(B, T, V), aux[0, 0]
